# Initial kernel scaffold; baseline (speedup 1.0000x reference)
import jax
import jax.numpy as jnp
from jax.experimental import pallas as pl


def kernel(f, b):
    raise NotImplementedError("write your pallas kernel here")


# speedup vs baseline: 1.0208x; 1.0208x over previous
import functools

import jax
import jax.numpy as jnp
from jax.experimental import pallas as pl
from jax.experimental.pallas import tpu as pltpu


def _im2col(x, k, stride):
    B, C, H, W = x.shape
    ho = -(-H // stride)
    wo = -(-W // stride)
    ph = max(0, (ho - 1) * stride + k - H)
    pw = max(0, (wo - 1) * stride + k - W)
    xp = jnp.pad(x, ((0, 0), (0, 0), (ph // 2, ph - ph // 2), (pw // 2, pw - pw // 2)))
    taps = []
    for dy in range(k):
        for dx in range(k):
            taps.append(xp[:, :, dy:dy + stride * (ho - 1) + 1:stride,
                               dx:dx + stride * (wo - 1) + 1:stride].reshape(B, C, ho * wo))
    return jnp.stack(taps, axis=2).reshape(B, C * k * k, ho * wo), ho, wo


def _overlap_add(pc, C, Hf, Wf, rate):
    B = pc.shape[0]
    pc8 = pc.reshape(B, Hf, Wf, C, 2, rate, 2, rate)
    acc = jnp.zeros((B, Hf + 1, Wf + 1, C, rate, rate), jnp.float32)
    for qy in range(2):
        for qx in range(2):
            acc = acc + jnp.pad(pc8[:, :, :, :, qy, :, qx, :],
                                ((0, 0), (qy, 1 - qy), (qx, 1 - qx),
                                 (0, 0), (0, 0), (0, 0)))
    out = jnp.transpose(acc, (0, 3, 1, 4, 2, 5))
    out = out.reshape(B, C, (Hf + 1) * rate, (Wf + 1) * rate)
    return out[:, :, 1:-1, 1:-1] * 0.25


def _shift_primed_minus(a, Wp, Wl):
    P, L = a.shape
    rows = jnp.concatenate(
        [jnp.zeros((1, L), a.dtype), a[P - Wp:P - 1, :], a[:P - Wp, :]], axis=0)
    return jnp.concatenate(
        [jnp.zeros((P, 1), a.dtype), rows[:, L - Wl:L - 1], rows[:, :L - Wl]], axis=1)


def _shift_primed_plus(a, Wp, Wl):
    P, L = a.shape
    rows = jnp.concatenate(
        [a[Wp:, :], a[1:Wp, :], jnp.zeros((1, L), a.dtype)], axis=0)
    return jnp.concatenate(
        [rows[:, Wl:], rows[:, 1:Wl], jnp.zeros((P, 1), a.dtype)], axis=1)


def _attn_body(x_ref, w_ref, raw_ref, o_ref, spad_ref, *, scale, Wf, Wb):
    CK, P = x_ref.shape
    L = w_ref.shape[1]

    w = w_ref[...]
    w32 = w.astype(jnp.float32)
    ssq = jnp.sum(w32 * w32, axis=0, keepdims=True)
    inv_nrm = 1.0 / jnp.maximum(jnp.sqrt(ssq), jnp.float32(1e-4))

    s = jax.lax.dot_general(x_ref[...], w, (((0,), (0,)), ((), ())),
                            preferred_element_type=jnp.float32)
    s = s * inv_nrm

    spad_ref[...] = jnp.zeros_like(spad_ref)
    spad_ref[pl.ds(8, P), pl.ds(128, L)] = s
    t = (s + spad_ref[pl.ds(7, P), pl.ds(127, L)]
           + spad_ref[pl.ds(9, P), pl.ds(129, L)])

    u = t + _shift_primed_minus(t, Wf, Wb) + _shift_primed_plus(t, Wf, Wb)

    logits = u * jnp.float32(scale)
    mx = jnp.max(logits, axis=1, keepdims=True)
    e = jnp.exp(logits - mx)
    denom = jnp.sum(e, axis=1, keepdims=True)
    prob = (e * (1.0 / denom)).astype(jnp.bfloat16)

    o_ref[...] = jax.lax.dot_general(prob, raw_ref[...], (((1,), (1,)), ((), ())),
                                     preferred_element_type=jnp.float32)


def _attention(x_cols, w_cols, raw_cols, scale, Wf, Wb):
    B, CK, P = x_cols.shape
    L = w_cols.shape[2]
    M = raw_cols.shape[1]
    body = functools.partial(_attn_body, scale=float(scale), Wf=Wf, Wb=Wb)
    return pl.pallas_call(
        body,
        out_shape=jax.ShapeDtypeStruct((B, P, M), jnp.float32),
        grid=(B,),
        in_specs=[pl.BlockSpec((None, CK, P), lambda i: (i, 0, 0)),
                  pl.BlockSpec((None, CK, L), lambda i: (i, 0, 0)),
                  pl.BlockSpec((None, M, L), lambda i: (i, 0, 0))],
        out_specs=pl.BlockSpec((None, P, M), lambda i: (i, 0, 0)),
        scratch_shapes=[pltpu.VMEM((P + 16, L + 256), jnp.float32)],
        compiler_params=pltpu.CompilerParams(dimension_semantics=("parallel",)),
    )(x_cols, w_cols, raw_cols)


def kernel(f, b):
    rate = 2
    ksize = 3
    scale = 10.0
    B, C, _, _ = f.shape

    raw_cols, _, _ = _im2col(b, 2 * rate, rate)
    fd = f[:, :, ::rate, ::rate]
    bd = b[:, :, ::rate, ::rate]
    Hf, Wf = fd.shape[2], fd.shape[3]
    Hb, Wb = bd.shape[2], bd.shape[3]

    w_cols, _, _ = _im2col(bd, ksize, 1)
    x_cols, _, _ = _im2col(fd, ksize, 1)

    pc = _attention(x_cols.astype(jnp.bfloat16),
                    w_cols.astype(jnp.bfloat16),
                    raw_cols.astype(jnp.bfloat16), scale, Wf, Wb)

    return _overlap_add(pc, C, Hf, Wf, rate)


# speedup vs baseline: 3.2612x; 3.1947x over previous
import functools

import jax
import jax.numpy as jnp
from jax.experimental import pallas as pl
from jax.experimental.pallas import tpu as pltpu


def _lane_bits(n_lanes):
    return jax.lax.broadcasted_iota(jnp.int32, (1, 1, n_lanes), 2)


def _sub_bits(n_rows):
    return jax.lax.broadcasted_iota(jnp.int32, (1, n_rows, 1), 1)


def _pack_even(x, block):
    n = x.shape[-1]
    lam = _lane_bits(n)
    for k in range((block // 2).bit_length() - 1):
        m = (((lam >> k) & 1) == 1) & (((lam >> (k + 1)) & 1) == 0)
        x = jnp.where(m, pltpu.roll(x, n - (1 << k), axis=2), x)
    return x


def _spread_even(x, block):
    n = x.shape[-1]
    lam = _lane_bits(n)
    for k in reversed(range((block // 2).bit_length() - 1)):
        m = (((lam >> k) & 1) == 0) & (((lam >> (k + 1)) & 1) == 1)
        x = jnp.where(m, pltpu.roll(x, 1 << k, axis=2), x)
    return x


def _pack_even_rows(x):
    r = x.shape[1]
    sub = _sub_bits(r)
    for k in range((r // 2).bit_length() - 1):
        m = (((sub >> k) & 1) == 1) & (((sub >> (k + 1)) & 1) == 0)
        x = jnp.where(m, pltpu.roll(x, r - (1 << k), axis=1), x)
    return x


def _spread_even_rows(x):
    r = x.shape[1]
    sub = _sub_bits(r)
    for k in reversed(range((r // 2).bit_length() - 1)):
        m = (((sub >> k) & 1) == 0) & (((sub >> (k + 1)) & 1) == 1)
        x = jnp.where(m, pltpu.roll(x, 1 << k, axis=1), x)
    return x


def _shift_primed_minus(a, Wp, Wl):
    P, L = a.shape
    rows = jnp.concatenate(
        [jnp.zeros((1, L), a.dtype), a[P - Wp:P - 1, :], a[:P - Wp, :]], axis=0)
    return jnp.concatenate(
        [jnp.zeros((P, 1), a.dtype), rows[:, L - Wl:L - 1], rows[:, :L - Wl]], axis=1)


def _shift_primed_plus(a, Wp, Wl):
    P, L = a.shape
    rows = jnp.concatenate(
        [a[Wp:, :], a[1:Wp, :], jnp.zeros((1, L), a.dtype)], axis=0)
    return jnp.concatenate(
        [rows[:, Wl:], rows[:, 1:Wl], jnp.zeros((P, 1), a.dtype)], axis=1)


def _grid_mask(H, W, a, b):
    li = jax.lax.broadcasted_iota(jnp.int32, (1, H * W), 1)
    h = li // W
    w = li % W
    return ((h + a >= 0) & (h + a < H) & (w + b >= 0) & (w + b < W))


def _tap(qflat, a, b, H, W):
    o = (-(a * W + b)) % (H * W)
    r = pltpu.roll(qflat, o, axis=1) if o else qflat
    return jnp.where(_grid_mask(H, W, a, b), r, jnp.zeros_like(r))


def _quads(ref, Hh, Wh, both):
    C, H, W = ref.shape
    x3 = ref[...].astype(jnp.bfloat16)

    def _q(h_par, rx):
        x = h_par if rx == 0 else pltpu.roll(h_par, W - 1, axis=2)
        return _pack_even(x, W)[:, :, :Wh].reshape(C, Hh * Wh)

    evr = _pack_even_rows(x3)[:, :Hh, :]
    if not both:
        return [_q(evr, 0)]
    odr = _pack_even_rows(pltpu.roll(x3, H - 1, axis=1))[:, :Hh, :]
    return [_q(evr, 0), _q(evr, 1), _q(odr, 0), _q(odr, 1)]


def _scores_body(f_ref, b_ref, prob_ref, quads_ref, spad_ref, *, scale, Hf, Wf):
    C = f_ref.shape[0]
    Hb, Wb = Hf, Wf
    P, L = Hf * Wf, Hb * Wb

    bq4 = _quads(b_ref, Hb, Wb, True)
    fq = _quads(f_ref, Hf, Wf, False)[0]
    quads_ref[...] = jnp.concatenate(bq4, axis=0)

    x_cols = jnp.concatenate(
        [_tap(fq, dy - 1, dx - 1, Hf, Wf) for dy in range(3) for dx in range(3)],
        axis=0)
    w_cols = jnp.concatenate(
        [_tap(bq4[0], dy - 1, dx - 1, Hb, Wb) for dy in range(3) for dx in range(3)],
        axis=0)

    w32 = w_cols.astype(jnp.float32)
    ssq = jnp.sum(w32 * w32, axis=0, keepdims=True)
    inv_nrm = 1.0 / jnp.maximum(jnp.sqrt(ssq), jnp.float32(1e-4))
    s = jax.lax.dot_general(x_cols, w_cols, (((0,), (0,)), ((), ())),
                            preferred_element_type=jnp.float32) * inv_nrm

    spad_ref[...] = jnp.zeros_like(spad_ref)
    spad_ref[pl.ds(8, P), pl.ds(128, L)] = s
    t = (s + spad_ref[pl.ds(7, P), pl.ds(127, L)]
           + spad_ref[pl.ds(9, P), pl.ds(129, L)])
    u = t + _shift_primed_minus(t, Wf, Wb) + _shift_primed_plus(t, Wf, Wb)

    logits = u * jnp.float32(scale)
    mx = jnp.max(logits, axis=1, keepdims=True)
    e = jnp.exp(logits - mx)
    denom = jnp.sum(e, axis=1, keepdims=True)
    prob_ref[...] = (e * (1.0 / denom)).astype(jnp.bfloat16)


def _attend_body(prob_ref, quads_ref, o_ref, *, Hf, Wf):
    C = quads_ref.shape[0] // 4
    Hb, Wb = Hf, Wf
    H, W = 2 * Hf, 2 * Wf
    prob = prob_ref[...]
    bq = [[quads_ref[pl.ds(0 * C, C), :], quads_ref[pl.ds(1 * C, C), :]],
          [quads_ref[pl.ds(2 * C, C), :], quads_ref[pl.ds(3 * C, C), :]]]

    amap = {0: (-1, 1), 1: (0, 0), 2: (0, 1), 3: (1, 0)}
    planes = [[None, None], [None, None]]
    for dy in range(4):
        ay, ry = amap[dy]
        for dx in range(4):
            ax, rx = amap[dx]
            slab = _tap(bq[ry][rx], ay, ax, Hb, Wb)
            pc = jax.lax.dot_general(slab, prob, (((1,), (1,)), ((), ())),
                                     preferred_element_type=jnp.float32)
            contrib = _tap(pc, -ay, -ax, Hf, Wf)
            prev = planes[ry][rx]
            planes[ry][rx] = contrib if prev is None else prev + contrib

    lam = _lane_bits(W)
    rows = []
    for ry in (0, 1):
        d = []
        for rx in (0, 1):
            p3 = (planes[ry][rx] * jnp.float32(0.25)).reshape(C, Hf, Wf)
            canvas = jnp.concatenate([p3, jnp.zeros_like(p3)], axis=2)
            d.append(_spread_even(canvas, W))
        rows.append(jnp.where((lam & 1) == 0, d[0], pltpu.roll(d[1], 1, axis=2)))
    sub = _sub_bits(H)
    full = [_spread_even_rows(jnp.concatenate([r, jnp.zeros_like(r)], axis=1))
            for r in rows]
    o_ref[...] = jnp.where((sub & 1) == 0, full[0], pltpu.roll(full[1], 1, axis=1))


def kernel(f, b):
    B, C, H, W = f.shape
    Hf, Wf = H // 2, W // 2
    P = Hf * Wf
    L = P

    sbody = functools.partial(_scores_body, scale=10.0, Hf=Hf, Wf=Wf)
    prob, quads = pl.pallas_call(
        sbody,
        out_shape=(jax.ShapeDtypeStruct((B, P, L), jnp.bfloat16),
                   jax.ShapeDtypeStruct((B, 4 * C, L), jnp.bfloat16)),
        grid=(B,),
        in_specs=[pl.BlockSpec((None, C, H, W), lambda i: (i, 0, 0, 0)),
                  pl.BlockSpec((None, C, H, W), lambda i: (i, 0, 0, 0))],
        out_specs=(pl.BlockSpec((None, P, L), lambda i: (i, 0, 0)),
                   pl.BlockSpec((None, 4 * C, L), lambda i: (i, 0, 0))),
        scratch_shapes=[pltpu.VMEM((P + 16, L + 256), jnp.float32)],
        compiler_params=pltpu.CompilerParams(dimension_semantics=("parallel",)),
    )(f, b)

    abody = functools.partial(_attend_body, Hf=Hf, Wf=Wf)
    return pl.pallas_call(
        abody,
        out_shape=jax.ShapeDtypeStruct((B, C, H, W), jnp.float32),
        grid=(B,),
        in_specs=[pl.BlockSpec((None, P, L), lambda i: (i, 0, 0)),
                  pl.BlockSpec((None, 4 * C, L), lambda i: (i, 0, 0))],
        out_specs=pl.BlockSpec((None, C, H, W), lambda i: (i, 0, 0, 0)),
        compiler_params=pltpu.CompilerParams(dimension_semantics=("parallel",)),
    )(prob, quads)


# speedup vs baseline: 7.4856x; 2.2954x over previous
import functools

import jax
import jax.numpy as jnp
from jax.experimental import pallas as pl
from jax.experimental.pallas import tpu as pltpu


def _lane1(n):
    return jax.lax.broadcasted_iota(jnp.int32, (1, n), 1)


def _bit(v, k):
    return (v >> k) & 1


def _pack2(x, block):
    n = x.shape[-1]
    lam = _lane1(n)
    for k in range((block // 2).bit_length() - 1):
        m = (_bit(lam, k) == 1) & (_bit(lam, k + 1) == 0)
        x = jnp.where(m, pltpu.roll(x, n - (1 << k), axis=1), x)
    return x


def _spread2(x, block):
    n = x.shape[-1]
    lam = _lane1(n)
    for k in reversed(range((block // 2).bit_length() - 1)):
        m = (_bit(lam, k) == 0) & (_bit(lam, k + 1) == 1)
        x = jnp.where(m, pltpu.roll(x, 1 << k, axis=1), x)
    return x


def _pack4u(x, unit_lg, nval_lg):
    n = x.shape[-1]
    lam = _lane1(n)
    for k in range(nval_lg):
        m = (_bit(lam, unit_lg + k) == 1) & (_bit(lam, unit_lg + k + 1) == 0)
        x = jnp.where(m, pltpu.roll(x, n - 3 * (1 << (unit_lg + k)), axis=1), x)
    return x


def _spread4u(x, unit_lg, nval_lg):
    n = x.shape[-1]
    lam = _lane1(n)
    for k in reversed(range(nval_lg)):
        m = ((_bit(lam, unit_lg + k) == 0) & (_bit(lam, unit_lg + k + 1) == 0)
             & (_bit(lam, unit_lg + k + 2) == 1))
        x = jnp.where(m, pltpu.roll(x, 3 * (1 << (unit_lg + k)), axis=1), x)
    return x


def _shift_primed_minus(a, Wp, Wl):
    P, L = a.shape
    rows = jnp.concatenate(
        [jnp.zeros((1, L), a.dtype), a[P - Wp:P - 1, :], a[:P - Wp, :]], axis=0)
    return jnp.concatenate(
        [jnp.zeros((P, 1), a.dtype), rows[:, L - Wl:L - 1], rows[:, :L - Wl]], axis=1)


def _shift_primed_plus(a, Wp, Wl):
    P, L = a.shape
    rows = jnp.concatenate(
        [a[Wp:, :], a[1:Wp, :], jnp.zeros((1, L), a.dtype)], axis=0)
    return jnp.concatenate(
        [rows[:, Wl:], rows[:, 1:Wl], jnp.zeros((P, 1), a.dtype)], axis=1)


def _grid_mask(H, W, a, b):
    li = jax.lax.broadcasted_iota(jnp.int32, (1, H * W), 1)
    h = li // W
    w = li % W
    return ((h + a >= 0) & (h + a < H) & (w + b >= 0) & (w + b < W))


def _tap(qflat, a, b, H, W):
    o = (-(a * W + b)) % (H * W)
    r = pltpu.roll(qflat, o, axis=1) if o else qflat
    return jnp.where(_grid_mask(H, W, a, b), r, jnp.zeros_like(r))


def _quads(ref, Hh, Wh, both):
    C, H, W = ref.shape
    n = H * W
    L = Hh * Wh
    unit_lg = (Wh - 1).bit_length()
    nval_lg = (Hh - 1).bit_length()
    xf = ref[...].reshape(C, n).astype(jnp.bfloat16)

    rxs = (0,) if not both else (0, 1)
    rys = (0,) if not both else (0, 1)
    wps = [_pack2(xf if rx == 0 else pltpu.roll(xf, n - 1, axis=1), W)
           for rx in rxs]
    out = []
    for ry in rys:
        for rx in rxs:
            hsrc = wps[rx] if ry == 0 else pltpu.roll(wps[rx], n - W, axis=1)
            out.append(_pack4u(hsrc, unit_lg, nval_lg)[:, :L])
    return out


def _scores_body(f_ref, b_ref, prob_ref, quads_ref, spad_ref, *, scale, Hf, Wf):
    C = f_ref.shape[0]
    Hb, Wb = Hf, Wf
    P, L = Hf * Wf, Hb * Wb

    bq4 = _quads(b_ref, Hb, Wb, True)
    fq = _quads(f_ref, Hf, Wf, False)[0]
    quads_ref[...] = jnp.concatenate(bq4, axis=0)

    x_cols = jnp.concatenate(
        [_tap(fq, dy - 1, dx - 1, Hf, Wf) for dy in range(3) for dx in range(3)],
        axis=0)
    w_cols = jnp.concatenate(
        [_tap(bq4[0], dy - 1, dx - 1, Hb, Wb) for dy in range(3) for dx in range(3)],
        axis=0)

    w32 = w_cols.astype(jnp.float32)
    ssq = jnp.sum(w32 * w32, axis=0, keepdims=True)
    inv_nrm = 1.0 / jnp.maximum(jnp.sqrt(ssq), jnp.float32(1e-4))
    s = jax.lax.dot_general(x_cols, w_cols, (((0,), (0,)), ((), ())),
                            preferred_element_type=jnp.float32) * inv_nrm

    spad_ref[...] = jnp.zeros_like(spad_ref)
    spad_ref[pl.ds(8, P), pl.ds(128, L)] = s
    t = (s + spad_ref[pl.ds(7, P), pl.ds(127, L)]
           + spad_ref[pl.ds(9, P), pl.ds(129, L)])
    u = t + _shift_primed_minus(t, Wf, Wb) + _shift_primed_plus(t, Wf, Wb)

    logits = u * jnp.float32(scale)
    mx = jnp.max(logits, axis=1, keepdims=True)
    e = jnp.exp(logits - mx)
    denom = jnp.sum(e, axis=1, keepdims=True)
    prob_ref[...] = (e * (1.0 / denom)).astype(jnp.bfloat16)


def _attend_body(prob_ref, quads_ref, o_ref, *, Hf, Wf):
    C = quads_ref.shape[0] // 4
    Hb, Wb = Hf, Wf
    H, W = 2 * Hf, 2 * Wf
    prob = prob_ref[...]
    bq = [[quads_ref[pl.ds(0 * C, C), :], quads_ref[pl.ds(1 * C, C), :]],
          [quads_ref[pl.ds(2 * C, C), :], quads_ref[pl.ds(3 * C, C), :]]]

    amap = {0: (-1, 1), 1: (0, 0), 2: (0, 1), 3: (1, 0)}
    planes = [[None, None], [None, None]]
    for dy in range(4):
        ay, ry = amap[dy]
        for dx in range(4):
            ax, rx = amap[dx]
            slab = _tap(bq[ry][rx], ay, ax, Hb, Wb)
            pc = jax.lax.dot_general(slab, prob, (((1,), (1,)), ((), ())),
                                     preferred_element_type=jnp.float32)
            contrib = _tap(pc, -ay, -ax, Hf, Wf)
            prev = planes[ry][rx]
            planes[ry][rx] = contrib if prev is None else prev + contrib

    n = H * W
    L4 = Hb * Wb
    unit_lg = (Wb - 1).bit_length()
    nval_lg = (Hb - 1).bit_length()
    lam = _lane1(n)
    unit_ok = ((lam >> unit_lg) & 3) == 0
    cols = []
    for rx in (0, 1):
        acc = None
        for ry in (0, 1):
            p = planes[ry][rx] * jnp.float32(0.25)
            canvas = jnp.concatenate(
                [p, jnp.zeros((C, n - L4), p.dtype)], axis=1)
            d = _spread4u(canvas, unit_lg, nval_lg)
            d = jnp.where(unit_ok, d, jnp.zeros_like(d))
            if ry:
                d = pltpu.roll(d, W, axis=1)
            acc = d if acc is None else acc + d
        cols.append(_spread2(acc, W))
    y = jnp.where((lam & 1) == 0, cols[0], pltpu.roll(cols[1], 1, axis=1))
    o_ref[...] = y.reshape(C, H, W)


def kernel(f, b):
    B, C, H, W = f.shape
    Hf, Wf = H // 2, W // 2
    P = Hf * Wf
    L = P

    sbody = functools.partial(_scores_body, scale=10.0, Hf=Hf, Wf=Wf)
    prob, quads = pl.pallas_call(
        sbody,
        out_shape=(jax.ShapeDtypeStruct((B, P, L), jnp.bfloat16),
                   jax.ShapeDtypeStruct((B, 4 * C, L), jnp.bfloat16)),
        grid=(B,),
        in_specs=[pl.BlockSpec((None, C, H, W), lambda i: (i, 0, 0, 0)),
                  pl.BlockSpec((None, C, H, W), lambda i: (i, 0, 0, 0))],
        out_specs=(pl.BlockSpec((None, P, L), lambda i: (i, 0, 0)),
                   pl.BlockSpec((None, 4 * C, L), lambda i: (i, 0, 0))),
        scratch_shapes=[pltpu.VMEM((P + 16, L + 256), jnp.float32)],
        compiler_params=pltpu.CompilerParams(dimension_semantics=("parallel",)),
    )(f, b)

    abody = functools.partial(_attend_body, Hf=Hf, Wf=Wf)
    return pl.pallas_call(
        abody,
        out_shape=jax.ShapeDtypeStruct((B, C, H, W), jnp.float32),
        grid=(B,),
        in_specs=[pl.BlockSpec((None, P, L), lambda i: (i, 0, 0)),
                  pl.BlockSpec((None, 4 * C, L), lambda i: (i, 0, 0))],
        out_specs=pl.BlockSpec((None, C, H, W), lambda i: (i, 0, 0, 0)),
        compiler_params=pltpu.CompilerParams(dimension_semantics=("parallel",)),
    )(prob, quads)


# speedup vs baseline: 8.1012x; 1.0822x over previous
import functools

import jax
import jax.numpy as jnp
from jax.experimental import pallas as pl
from jax.experimental.pallas import tpu as pltpu


def _lane1(n):
    return jax.lax.broadcasted_iota(jnp.int32, (1, n), 1)


def _bit(v, k):
    return (v >> k) & 1


def _pack2(x, block):
    n = x.shape[-1]
    lam = _lane1(n)
    for k in range((block // 2).bit_length() - 1):
        m = (_bit(lam, k) == 1) & (_bit(lam, k + 1) == 0)
        x = jnp.where(m, pltpu.roll(x, n - (1 << k), axis=1), x)
    return x


def _spread2(x, block):
    n = x.shape[-1]
    lam = _lane1(n)
    for k in reversed(range((block // 2).bit_length() - 1)):
        m = (_bit(lam, k) == 0) & (_bit(lam, k + 1) == 1)
        x = jnp.where(m, pltpu.roll(x, 1 << k, axis=1), x)
    return x


def _pack4u(x, unit_lg, nval_lg):
    n = x.shape[-1]
    lam = _lane1(n)
    for k in range(nval_lg):
        m = (_bit(lam, unit_lg + k) == 1) & (_bit(lam, unit_lg + k + 1) == 0)
        x = jnp.where(m, pltpu.roll(x, n - 3 * (1 << (unit_lg + k)), axis=1), x)
    return x


def _spread4u(x, unit_lg, nval_lg):
    n = x.shape[-1]
    lam = _lane1(n)
    for k in reversed(range(nval_lg)):
        m = ((_bit(lam, unit_lg + k) == 0) & (_bit(lam, unit_lg + k + 1) == 0)
             & (_bit(lam, unit_lg + k + 2) == 1))
        x = jnp.where(m, pltpu.roll(x, 3 * (1 << (unit_lg + k)), axis=1), x)
    return x


def _shift_primed_minus(a, Wp, Wl):
    P, L = a.shape
    rows = jnp.concatenate(
        [jnp.zeros((1, L), a.dtype), a[P - Wp:P - 1, :], a[:P - Wp, :]], axis=0)
    return jnp.concatenate(
        [jnp.zeros((P, 1), a.dtype), rows[:, L - Wl:L - 1], rows[:, :L - Wl]], axis=1)


def _shift_primed_plus(a, Wp, Wl):
    P, L = a.shape
    rows = jnp.concatenate(
        [a[Wp:, :], a[1:Wp, :], jnp.zeros((1, L), a.dtype)], axis=0)
    return jnp.concatenate(
        [rows[:, Wl:], rows[:, 1:Wl], jnp.zeros((P, 1), a.dtype)], axis=1)


def _grid_mask(H, W, a, b):
    li = jax.lax.broadcasted_iota(jnp.int32, (1, H * W), 1)
    h = li // W
    w = li % W
    return ((h + a >= 0) & (h + a < H) & (w + b >= 0) & (w + b < W))


def _tap(qflat, a, b, H, W):
    o = (-(a * W + b)) % (H * W)
    r = pltpu.roll(qflat, o, axis=1) if o else qflat
    return jnp.where(_grid_mask(H, W, a, b), r, jnp.zeros_like(r))


def _quads(ref, Hh, Wh, both):
    C, H, W = ref.shape
    n = H * W
    L = Hh * Wh
    unit_lg = (Wh - 1).bit_length()
    nval_lg = (Hh - 1).bit_length()
    xf = ref[...].reshape(C, n).astype(jnp.bfloat16)

    rxs = (0,) if not both else (0, 1)
    rys = (0,) if not both else (0, 1)
    wps = [_pack2(xf if rx == 0 else pltpu.roll(xf, n - 1, axis=1), W)
           for rx in rxs]
    out = []
    for ry in rys:
        for rx in rxs:
            hsrc = wps[rx] if ry == 0 else pltpu.roll(wps[rx], n - W, axis=1)
            out.append(_pack4u(hsrc, unit_lg, nval_lg)[:, :L])
    return out


def _scores_body(f_ref, b_ref, prob_ref, quads_ref, spad_ref, *, scale, Hf, Wf):
    C = f_ref.shape[0]
    Hb, Wb = Hf, Wf
    P, L = Hf * Wf, Hb * Wb

    bq4 = _quads(b_ref, Hb, Wb, True)
    fq = _quads(f_ref, Hf, Wf, False)[0]
    quads_ref[...] = jnp.concatenate(bq4, axis=0)

    x_cols = jnp.concatenate(
        [_tap(fq, dy - 1, dx - 1, Hf, Wf) for dy in range(3) for dx in range(3)],
        axis=0)
    w_cols = jnp.concatenate(
        [_tap(bq4[0], dy - 1, dx - 1, Hb, Wb) for dy in range(3) for dx in range(3)],
        axis=0)

    w32 = w_cols.astype(jnp.float32)
    ssq = jnp.sum(w32 * w32, axis=0, keepdims=True)
    inv_nrm = 1.0 / jnp.maximum(jnp.sqrt(ssq), jnp.float32(1e-4))
    s = jax.lax.dot_general(x_cols, w_cols, (((0,), (0,)), ((), ())),
                            preferred_element_type=jnp.float32) * inv_nrm

    spad_ref[...] = jnp.zeros_like(spad_ref)
    spad_ref[pl.ds(8, P), pl.ds(128, L)] = s
    t = (s + spad_ref[pl.ds(7, P), pl.ds(127, L)]
           + spad_ref[pl.ds(9, P), pl.ds(129, L)])
    u = t + _shift_primed_minus(t, Wf, Wb) + _shift_primed_plus(t, Wf, Wb)

    logits = u * jnp.float32(scale)
    mx = jnp.max(logits, axis=1, keepdims=True)
    e = jnp.exp(logits - mx)
    denom = jnp.sum(e, axis=1, keepdims=True)
    prob_ref[...] = (e * (1.0 / denom)).astype(jnp.bfloat16)


def _attend_body(prob_ref, quads_ref, o_ref, *, Hf, Wf):
    C = quads_ref.shape[0] // 4
    Hb, Wb = Hf, Wf
    H, W = 2 * Hf, 2 * Wf
    prob = prob_ref[...]
    bq = [[quads_ref[pl.ds(0 * C, C), :], quads_ref[pl.ds(1 * C, C), :]],
          [quads_ref[pl.ds(2 * C, C), :], quads_ref[pl.ds(3 * C, C), :]]]

    amap = {0: (-1, 1), 1: (0, 0), 2: (0, 1), 3: (1, 0)}
    taps = [(amap[dy], amap[dx]) for dy in range(4) for dx in range(4)]
    slabs = jnp.concatenate(
        [_tap(bq[ry][rx], ay, ax, Hb, Wb) for (ay, ry), (ax, rx) in taps],
        axis=0)
    pc_all = jax.lax.dot_general(slabs, prob, (((1,), (1,)), ((), ())),
                                 preferred_element_type=jnp.float32)
    planes = [[None, None], [None, None]]
    for i, ((ay, ry), (ax, rx)) in enumerate(taps):
        contrib = _tap(pc_all[i * C:(i + 1) * C, :], -ay, -ax, Hf, Wf)
        prev = planes[ry][rx]
        planes[ry][rx] = contrib if prev is None else prev + contrib

    n = H * W
    L4 = Hb * Wb
    unit_lg = (Wb - 1).bit_length()
    nval_lg = (Hb - 1).bit_length()
    lam = _lane1(n)
    unit_ok = ((lam >> unit_lg) & 3) == 0
    cols = []
    for rx in (0, 1):
        acc = None
        for ry in (0, 1):
            p = planes[ry][rx] * jnp.float32(0.25)
            canvas = jnp.concatenate(
                [p, jnp.zeros((C, n - L4), p.dtype)], axis=1)
            d = _spread4u(canvas, unit_lg, nval_lg)
            d = jnp.where(unit_ok, d, jnp.zeros_like(d))
            if ry:
                d = pltpu.roll(d, W, axis=1)
            acc = d if acc is None else acc + d
        cols.append(_spread2(acc, W))
    y = jnp.where((lam & 1) == 0, cols[0], pltpu.roll(cols[1], 1, axis=1))
    o_ref[...] = y.reshape(C, H, W)


def kernel(f, b):
    B, C, H, W = f.shape
    Hf, Wf = H // 2, W // 2
    P = Hf * Wf
    L = P

    sbody = functools.partial(_scores_body, scale=10.0, Hf=Hf, Wf=Wf)
    prob, quads = pl.pallas_call(
        sbody,
        out_shape=(jax.ShapeDtypeStruct((B, P, L), jnp.bfloat16),
                   jax.ShapeDtypeStruct((B, 4 * C, L), jnp.bfloat16)),
        grid=(B,),
        in_specs=[pl.BlockSpec((None, C, H, W), lambda i: (i, 0, 0, 0)),
                  pl.BlockSpec((None, C, H, W), lambda i: (i, 0, 0, 0))],
        out_specs=(pl.BlockSpec((None, P, L), lambda i: (i, 0, 0)),
                   pl.BlockSpec((None, 4 * C, L), lambda i: (i, 0, 0))),
        scratch_shapes=[pltpu.VMEM((P + 16, L + 256), jnp.float32)],
        compiler_params=pltpu.CompilerParams(dimension_semantics=("parallel",)),
    )(f, b)

    abody = functools.partial(_attend_body, Hf=Hf, Wf=Wf)
    return pl.pallas_call(
        abody,
        out_shape=jax.ShapeDtypeStruct((B, C, H, W), jnp.float32),
        grid=(B,),
        in_specs=[pl.BlockSpec((None, P, L), lambda i: (i, 0, 0)),
                  pl.BlockSpec((None, 4 * C, L), lambda i: (i, 0, 0))],
        out_specs=pl.BlockSpec((None, C, H, W), lambda i: (i, 0, 0, 0)),
        compiler_params=pltpu.CompilerParams(dimension_semantics=("parallel",)),
    )(prob, quads)


# speedup vs baseline: 9.2135x; 1.1373x over previous
import functools

import jax
import jax.numpy as jnp
from jax.experimental import pallas as pl
from jax.experimental.pallas import tpu as pltpu


def _lane1(n):
    return jax.lax.broadcasted_iota(jnp.int32, (1, n), 1)


def _bit(v, k):
    return (v >> k) & 1


def _pack2(x, block):
    n = x.shape[-1]
    lam = _lane1(n)
    for k in range((block // 2).bit_length() - 1):
        m = (_bit(lam, k) == 1) & (_bit(lam, k + 1) == 0)
        x = jnp.where(m, pltpu.roll(x, n - (1 << k), axis=1), x)
    return x


def _spread2(x, block):
    n = x.shape[-1]
    lam = _lane1(n)
    for k in reversed(range((block // 2).bit_length() - 1)):
        m = (_bit(lam, k) == 0) & (_bit(lam, k + 1) == 1)
        x = jnp.where(m, pltpu.roll(x, 1 << k, axis=1), x)
    return x


def _pack4u(x, unit_lg, nval_lg):
    n = x.shape[-1]
    lam = _lane1(n)
    for k in range(nval_lg):
        m = (_bit(lam, unit_lg + k) == 1) & (_bit(lam, unit_lg + k + 1) == 0)
        x = jnp.where(m, pltpu.roll(x, n - 3 * (1 << (unit_lg + k)), axis=1), x)
    return x


def _spread4u(x, unit_lg, nval_lg):
    n = x.shape[-1]
    lam = _lane1(n)
    for k in reversed(range(nval_lg)):
        m = ((_bit(lam, unit_lg + k) == 0) & (_bit(lam, unit_lg + k + 1) == 0)
             & (_bit(lam, unit_lg + k + 2) == 1))
        x = jnp.where(m, pltpu.roll(x, 3 * (1 << (unit_lg + k)), axis=1), x)
    return x


def _shift_primed_minus(a, Wp, Wl):
    P, L = a.shape
    rows = jnp.concatenate(
        [jnp.zeros((1, L), a.dtype), a[P - Wp:P - 1, :], a[:P - Wp, :]], axis=0)
    return jnp.concatenate(
        [jnp.zeros((P, 1), a.dtype), rows[:, L - Wl:L - 1], rows[:, :L - Wl]], axis=1)


def _shift_primed_plus(a, Wp, Wl):
    P, L = a.shape
    rows = jnp.concatenate(
        [a[Wp:, :], a[1:Wp, :], jnp.zeros((1, L), a.dtype)], axis=0)
    return jnp.concatenate(
        [rows[:, Wl:], rows[:, 1:Wl], jnp.zeros((P, 1), a.dtype)], axis=1)


def _grid_mask(H, W, a, b):
    li = jax.lax.broadcasted_iota(jnp.int32, (1, H * W), 1)
    h = li // W
    w = li % W
    return ((h + a >= 0) & (h + a < H) & (w + b >= 0) & (w + b < W))


def _tap(qflat, a, b, H, W):
    o = (-(a * W + b)) % (H * W)
    r = pltpu.roll(qflat, o, axis=1) if o else qflat
    return jnp.where(_grid_mask(H, W, a, b), r, jnp.zeros_like(r))


def _quads(ref, Hh, Wh, both):
    C, H, W = ref.shape
    n = H * W
    L = Hh * Wh
    unit_lg = (Wh - 1).bit_length()
    nval_lg = (Hh - 1).bit_length()
    xf = ref[...].reshape(C, n).astype(jnp.bfloat16)

    rxs = (0,) if not both else (0, 1)
    rys = (0,) if not both else (0, 1)
    wps = [_pack2(xf if rx == 0 else pltpu.roll(xf, n - 1, axis=1), W)
           for rx in rxs]
    out = []
    for ry in rys:
        for rx in rxs:
            hsrc = wps[rx] if ry == 0 else pltpu.roll(wps[rx], n - W, axis=1)
            out.append(_pack4u(hsrc, unit_lg, nval_lg)[:, :L])
    return out


def _scores_body(f_ref, b_ref, prob_ref, quads_ref, spad_ref, *, scale, Hf, Wf):
    C = f_ref.shape[0]
    Hb, Wb = Hf, Wf
    P, L = Hf * Wf, Hb * Wb

    bq4 = _quads(b_ref, Hb, Wb, True)
    fq = _quads(f_ref, Hf, Wf, False)[0]
    quads_ref[...] = jnp.concatenate(bq4, axis=0)

    x_cols = jnp.concatenate(
        [_tap(fq, dy - 1, dx - 1, Hf, Wf) for dy in range(3) for dx in range(3)],
        axis=0)
    w_cols = jnp.concatenate(
        [_tap(bq4[0], dy - 1, dx - 1, Hb, Wb) for dy in range(3) for dx in range(3)],
        axis=0)

    w32 = w_cols.astype(jnp.float32)
    ssq = jnp.sum(w32 * w32, axis=0, keepdims=True)
    inv_nrm = 1.0 / jnp.maximum(jnp.sqrt(ssq), jnp.float32(1e-4))
    s = jax.lax.dot_general(x_cols, w_cols, (((0,), (0,)), ((), ())),
                            preferred_element_type=jnp.float32) * inv_nrm

    spad_ref[...] = jnp.zeros_like(spad_ref)
    spad_ref[pl.ds(8, P), pl.ds(128, L)] = s
    t = (s + spad_ref[pl.ds(7, P), pl.ds(127, L)]
           + spad_ref[pl.ds(9, P), pl.ds(129, L)])
    u = t + _shift_primed_minus(t, Wf, Wb) + _shift_primed_plus(t, Wf, Wb)

    logits = u * jnp.float32(scale)
    mx = jnp.max(logits, axis=1, keepdims=True)
    e = jnp.exp(logits - mx)
    denom = jnp.sum(e, axis=1, keepdims=True)
    prob_ref[...] = (e * (1.0 / denom)).astype(jnp.bfloat16)


def _attend_body(prob_ref, quads_ref, o_ref, *, Hf, Wf):
    C = quads_ref.shape[0] // 4
    Hb, Wb = Hf, Wf
    H, W = 2 * Hf, 2 * Wf
    prob = prob_ref[...]
    bq = [[quads_ref[pl.ds(0 * C, C), :], quads_ref[pl.ds(1 * C, C), :]],
          [quads_ref[pl.ds(2 * C, C), :], quads_ref[pl.ds(3 * C, C), :]]]

    amap = {0: (-1, 1), 1: (0, 0), 2: (0, 1), 3: (1, 0)}
    taps = [(amap[dy], amap[dx]) for dy in range(4) for dx in range(4)]
    slabs = jnp.concatenate(
        [_tap(bq[ry][rx], ay, ax, Hb, Wb) for (ay, ry), (ax, rx) in taps],
        axis=0)
    pc_all = jax.lax.dot_general(slabs, prob, (((1,), (1,)), ((), ())),
                                 preferred_element_type=jnp.float32)
    planes = [[None, None], [None, None]]
    for i, ((ay, ry), (ax, rx)) in enumerate(taps):
        contrib = _tap(pc_all[i * C:(i + 1) * C, :], -ay, -ax, Hf, Wf)
        prev = planes[ry][rx]
        planes[ry][rx] = contrib if prev is None else prev + contrib

    n = H * W
    L4 = Hb * Wb
    unit_lg = (Wb - 1).bit_length()
    nval_lg = (Hb - 1).bit_length()
    lam = _lane1(n)
    unit_ok = ((lam >> unit_lg) & 3) == 0
    cols = []
    for rx in (0, 1):
        acc = None
        for ry in (0, 1):
            p = (planes[ry][rx] * jnp.float32(0.25)).astype(jnp.bfloat16)
            canvas = jnp.concatenate(
                [p, jnp.zeros((C, n - L4), p.dtype)], axis=1)
            d = _spread4u(canvas, unit_lg, nval_lg)
            d = jnp.where(unit_ok, d, jnp.zeros_like(d))
            if ry:
                d = pltpu.roll(d, W, axis=1)
            acc = d if acc is None else acc + d
        cols.append(_spread2(acc, W))
    y = jnp.where((lam & 1) == 0, cols[0], pltpu.roll(cols[1], 1, axis=1))
    o_ref[...] = y.astype(jnp.float32).reshape(C, H, W)


def kernel(f, b):
    B, C, H, W = f.shape
    Hf, Wf = H // 2, W // 2
    P = Hf * Wf
    L = P

    sbody = functools.partial(_scores_body, scale=10.0, Hf=Hf, Wf=Wf)
    prob, quads = pl.pallas_call(
        sbody,
        out_shape=(jax.ShapeDtypeStruct((B, P, L), jnp.bfloat16),
                   jax.ShapeDtypeStruct((B, 4 * C, L), jnp.bfloat16)),
        grid=(B,),
        in_specs=[pl.BlockSpec((None, C, H, W), lambda i: (i, 0, 0, 0)),
                  pl.BlockSpec((None, C, H, W), lambda i: (i, 0, 0, 0))],
        out_specs=(pl.BlockSpec((None, P, L), lambda i: (i, 0, 0)),
                   pl.BlockSpec((None, 4 * C, L), lambda i: (i, 0, 0))),
        scratch_shapes=[pltpu.VMEM((P + 16, L + 256), jnp.float32)],
        compiler_params=pltpu.CompilerParams(dimension_semantics=("parallel",)),
    )(f, b)

    abody = functools.partial(_attend_body, Hf=Hf, Wf=Wf)
    return pl.pallas_call(
        abody,
        out_shape=jax.ShapeDtypeStruct((B, C, H, W), jnp.float32),
        grid=(B,),
        in_specs=[pl.BlockSpec((None, P, L), lambda i: (i, 0, 0)),
                  pl.BlockSpec((None, 4 * C, L), lambda i: (i, 0, 0))],
        out_specs=pl.BlockSpec((None, C, H, W), lambda i: (i, 0, 0, 0)),
        compiler_params=pltpu.CompilerParams(dimension_semantics=("parallel",)),
    )(prob, quads)
